# baseline (speedup 1.0000x reference)
import functools

import jax
import jax.numpy as jnp
from jax import lax
from jax.experimental import pallas as pl
from jax.experimental.pallas import tpu as pltpu
from jax.experimental.pallas import tpu_sc as plsc

_NE = 1024
_D = 64
_BR = 1024
_NROWS = 32 * 1024


def _tc_body(x_ref, et_ref, idx_ref, loss_ref):
    i = pl.program_id(0)
    x = x_ref[...]
    ssq = jnp.sum(x * x, axis=1, keepdims=True)
    norm = jnp.sqrt(ssq)
    inv = 1.0 / jnp.maximum(norm, 1e-12)
    xn = x * inv

    et = et_ref[...]
    esq = jnp.sum(et * et, axis=0, keepdims=True)
    s = lax.dot_general(xn, et, (((1,), (0,)), ((), ())),
                        preferred_element_type=jnp.float32)
    g = esq - 2.0 * s
    m = jnp.min(g, axis=1, keepdims=True)
    col_ids = lax.broadcasted_iota(jnp.int32, g.shape, 1)
    idx = jnp.min(jnp.where(g == m, col_ids, _NE), axis=1, keepdims=True)
    idx_ref[...] = idx

    xnsq = ssq * (inv * inv)
    part = jnp.sum(xnsq + m) * (1.0 / (_NROWS * _D))

    @pl.when(i == 0)
    def _():
        loss_ref[0, 0] = 0.0

    loss_ref[0, 0] += part


_tc_call = pl.pallas_call(
    _tc_body,
    grid=(_NROWS // _BR,),
    in_specs=[
        pl.BlockSpec((_BR, _D), lambda i: (i, 0)),
        pl.BlockSpec((_D, _NE), lambda i: (0, 0)),
    ],
    out_specs=[
        pl.BlockSpec((_BR, 1), lambda i: (i, 0)),
        pl.BlockSpec(memory_space=pltpu.SMEM, block_shape=(1, 1),
                     index_map=lambda i: (0, 0)),
    ],
    out_shape=[
        jax.ShapeDtypeStruct((_NROWS, 1), jnp.int32),
        jax.ShapeDtypeStruct((1, 1), jnp.float32),
    ],
    compiler_params=pltpu.CompilerParams(dimension_semantics=("arbitrary",)),
)


_NC, _NS = 2, 16
_NW = _NC * _NS
_BPW = _NROWS // _NW


@functools.cache
def _sc_gather_call():
    @functools.partial(
        pl.kernel,
        mesh=plsc.VectorSubcoreMesh(core_axis_name="c", subcore_axis_name="s"),
        out_type=jax.ShapeDtypeStruct((_NROWS, _D), jnp.float32),
        scratch_types=[
            pltpu.VMEM((_BPW,), jnp.int32),
            pltpu.VMEM((_BPW, _D), jnp.float32),
            pltpu.SemaphoreType.DMA,
        ],
        compiler_params=pltpu.CompilerParams(use_tc_tiling_on_sc=False),
    )
    def _sc_gather(table_hbm, idx_hbm, out_hbm, idx_v, rows_v, sem):
        wid = lax.axis_index("s") * _NC + lax.axis_index("c")
        base = wid * _BPW
        pltpu.sync_copy(idx_hbm.at[pl.ds(base, _BPW)], idx_v)
        pltpu.async_copy(table_hbm.at[idx_v], rows_v, sem).wait()
        pltpu.sync_copy(rows_v, out_hbm.at[pl.ds(base, _BPW)])

    return _sc_gather


def kernel(inputs, embeddings):
    orig_shape = inputs.shape
    x2d = inputs.reshape(-1, _D)
    idx_col, loss11 = _tc_call(x2d, embeddings.T)
    idx_flat = idx_col.reshape(-1)
    q = _sc_gather_call()(embeddings, idx_flat)
    quantized = q.reshape(orig_shape)
    loss = loss11[0, 0]
    encoding_indices = idx_flat.reshape(orig_shape[:-1])
    return (quantized, loss, encoding_indices)


# speedup vs baseline: 1.2223x; 1.2223x over previous
import functools

import jax
import jax.numpy as jnp
from jax import lax
from jax.experimental import pallas as pl
from jax.experimental.pallas import tpu as pltpu
from jax.experimental.pallas import tpu_sc as plsc

_NE = 1024
_D = 64
_B = 32
_BR = 1024
_NROWS = _B * _BR
_NC, _NS = 2, 16
_L = 16
_DCH = 16


def _tc_body(xt_ref, e_ref, idx_ref, loss_ref):
    i = pl.program_id(0)
    xt = xt_ref[0]
    ssq = jnp.sum(xt * xt, axis=0, keepdims=True)
    norm = jnp.sqrt(ssq)
    inv = 1.0 / jnp.maximum(norm, 1e-12)
    xnt = xt * inv

    s = lax.dot_general(e_ref[...], xnt, (((1,), (0,)), ((), ())),
                        preferred_element_type=jnp.float32)
    m = jnp.max(s, axis=0, keepdims=True)
    row_ids = lax.broadcasted_iota(jnp.int32, s.shape, 0)
    idx = jnp.min(jnp.where(s == m, row_ids, _NE), axis=0, keepdims=True)
    idx_ref[0] = idx

    xnsq = ssq * (inv * inv)
    part = jnp.sum(xnsq - 2.0 * m + 1.0) * (1.0 / (_NROWS * _D))

    @pl.when(i == 0)
    def _():
        loss_ref[0, 0] = 0.0

    loss_ref[0, 0] += part


_tc_call = pl.pallas_call(
    _tc_body,
    grid=(_B,),
    in_specs=[
        pl.BlockSpec((1, _D, _BR), lambda i: (i, 0, 0)),
        pl.BlockSpec((_NE, _D), lambda i: (0, 0)),
    ],
    out_specs=[
        pl.BlockSpec((1, 1, _BR), lambda i: (i, 0, 0)),
        pl.BlockSpec(memory_space=pltpu.SMEM, block_shape=(1, 1),
                     index_map=lambda i: (0, 0)),
    ],
    out_shape=[
        jax.ShapeDtypeStruct((_B, 1, _BR), jnp.int32),
        jax.ShapeDtypeStruct((1, 1), jnp.float32),
    ],
    compiler_params=pltpu.CompilerParams(dimension_semantics=("arbitrary",)),
)


@functools.cache
def _sc_gather_call():
    @functools.partial(
        pl.kernel,
        mesh=plsc.VectorSubcoreMesh(core_axis_name="c", subcore_axis_name="s"),
        out_type=jax.ShapeDtypeStruct((_B, _D * _BR), jnp.float32),
        scratch_types=[
            pltpu.VMEM((_D, _NE), jnp.float32),
            pltpu.VMEM((_BR,), jnp.int32),
            pltpu.VMEM((_DCH * _BR,), jnp.float32),
        ],
        compiler_params=pltpu.CompilerParams(use_tc_tiling_on_sc=False,
                                             needs_layout_passes=False),
    )
    def _sc_gather(etf_hbm, idx_hbm, out_hbm, et_v, idx_v, q_v):
        b = lax.axis_index("s") * _NC + lax.axis_index("c")
        pltpu.sync_copy(etf_hbm, et_v)
        pltpu.sync_copy(idx_hbm.at[b], idx_v)

        for c in range(_D // _DCH):
            def body(r, carry, c=c):
                base = r * _L
                cols = idx_v[pl.ds(base, _L)]
                for dd in range(_DCH):
                    rows = jnp.full((_L,), c * _DCH + dd, jnp.int32)
                    q_v[pl.ds(dd * _BR + base, _L)] = plsc.load_gather(
                        et_v, [rows, cols])
                return carry

            lax.fori_loop(0, _BR // _L, body, 0)
            pltpu.sync_copy(q_v, out_hbm.at[b, pl.ds(c * _DCH * _BR,
                                                     _DCH * _BR)])

    return _sc_gather


def kernel(inputs, embeddings):
    orig_shape = inputs.shape
    xt3 = jnp.swapaxes(inputs, 1, 2)
    idx3, loss11 = _tc_call(xt3, embeddings)
    idx2d = idx3.reshape(_B, _BR)
    et = embeddings.T
    qt = _sc_gather_call()(et, idx2d)
    quantized = jnp.swapaxes(qt.reshape(_B, _D, _BR), 1, 2).reshape(orig_shape)
    loss = loss11[0, 0]
    encoding_indices = idx2d.reshape(orig_shape[:-1])
    return (quantized, loss, encoding_indices)


# speedup vs baseline: 1.3861x; 1.1340x over previous
import functools

import jax
import jax.numpy as jnp
from jax import lax
from jax.experimental import pallas as pl
from jax.experimental.pallas import tpu as pltpu
from jax.experimental.pallas import tpu_sc as plsc

_NE = 1024
_D = 64
_B = 32
_BR = 1024
_NROWS = _B * _BR
_NC, _NS = 2, 16
_L = 16
_DCH = 16


def _tc_body(xt_ref, e_ref, idx_ref, loss_ref):
    i = pl.program_id(0)
    xt = xt_ref[0]
    ssq = jnp.sum(xt * xt, axis=0, keepdims=True)
    norm = jnp.sqrt(ssq)
    inv = 1.0 / jnp.maximum(norm, 1e-12)
    xnt = xt * inv

    s = lax.dot_general(e_ref[...], xnt, (((1,), (0,)), ((), ())),
                        preferred_element_type=jnp.float32)
    m = jnp.max(s, axis=0, keepdims=True)
    row_ids = lax.broadcasted_iota(jnp.int32, s.shape, 0)
    idx = jnp.min(jnp.where(s == m, row_ids, _NE), axis=0, keepdims=True)
    idx_ref[0] = idx

    xnsq = ssq * (inv * inv)
    part = jnp.sum(xnsq - 2.0 * m + 1.0) * (1.0 / (_NROWS * _D))

    @pl.when(i == 0)
    def _():
        loss_ref[0, 0] = 0.0

    loss_ref[0, 0] += part


_tc_call = pl.pallas_call(
    _tc_body,
    grid=(_B,),
    in_specs=[
        pl.BlockSpec((1, _D, _BR), lambda i: (i, 0, 0)),
        pl.BlockSpec((_NE, _D), lambda i: (0, 0)),
    ],
    out_specs=[
        pl.BlockSpec((1, 1, _BR), lambda i: (i, 0, 0)),
        pl.BlockSpec(memory_space=pltpu.SMEM, block_shape=(1, 1),
                     index_map=lambda i: (0, 0)),
    ],
    out_shape=[
        jax.ShapeDtypeStruct((_B, 1, _BR), jnp.int32),
        jax.ShapeDtypeStruct((1, 1), jnp.float32),
    ],
    compiler_params=pltpu.CompilerParams(dimension_semantics=("arbitrary",)),
)


@functools.cache
def _sc_gather_call():
    @functools.partial(
        pl.kernel,
        mesh=plsc.VectorSubcoreMesh(core_axis_name="c", subcore_axis_name="s"),
        out_type=jax.ShapeDtypeStruct((_B, _D * _BR), jnp.float32),
        scratch_types=[
            pltpu.VMEM((_D, _NE), jnp.float32),
            pltpu.VMEM((_BR,), jnp.int32),
            pltpu.VMEM((_DCH * _BR,), jnp.float32),
        ],
        compiler_params=pltpu.CompilerParams(use_tc_tiling_on_sc=False,
                                             needs_layout_passes=False),
    )
    def _sc_gather(etf_hbm, idx_hbm, out_hbm, et_v, idx_v, q_v):
        b = lax.axis_index("s") * _NC + lax.axis_index("c")
        pltpu.sync_copy(etf_hbm, et_v)
        pltpu.sync_copy(idx_hbm.at[b], idx_v)

        for c in range(_D // _DCH):
            @plsc.parallel_loop(0, _BR // _L, 1, unroll=4)
            def _(r, c=c):
                base = r * _L
                cols = idx_v[pl.ds(base, _L)]
                for dd in range(_DCH):
                    rows = jnp.full((_L,), c * _DCH + dd, jnp.int32)
                    q_v[pl.ds(dd * _BR + base, _L)] = plsc.load_gather(
                        et_v, [rows, cols])

            pltpu.sync_copy(q_v, out_hbm.at[b, pl.ds(c * _DCH * _BR,
                                                     _DCH * _BR)])

    return _sc_gather


def kernel(inputs, embeddings):
    orig_shape = inputs.shape
    xt3 = jnp.swapaxes(inputs, 1, 2)
    idx3, loss11 = _tc_call(xt3, embeddings)
    idx2d = idx3.reshape(_B, _BR)
    et = embeddings.T
    qt = _sc_gather_call()(et, idx2d)
    quantized = jnp.swapaxes(qt.reshape(_B, _D, _BR), 1, 2).reshape(orig_shape)
    loss = loss11[0, 0]
    encoding_indices = idx2d.reshape(orig_shape[:-1])
    return (quantized, loss, encoding_indices)


# speedup vs baseline: 1.6159x; 1.1658x over previous
import functools

import jax
import jax.numpy as jnp
from jax import lax
from jax.experimental import pallas as pl
from jax.experimental.pallas import tpu as pltpu
from jax.experimental.pallas import tpu_sc as plsc

_NE = 1024
_D = 64
_B = 32
_BR = 1024
_NROWS = _B * _BR
_NC, _NS = 2, 16
_L = 16
_DCH = 16


def _tc_body(xt_ref, e_ref, idx_ref, loss_ref):
    i = pl.program_id(0)
    xt = xt_ref[0]
    ssq = jnp.sum(xt * xt, axis=0, keepdims=True)
    norm = jnp.sqrt(ssq)
    inv = 1.0 / jnp.maximum(norm, 1e-12)
    xnt = xt * inv

    s = lax.dot_general(e_ref[...], xnt, (((1,), (0,)), ((), ())),
                        preferred_element_type=jnp.float32)
    m = jnp.max(s, axis=0, keepdims=True)
    idx = jnp.argmax(s, axis=0)[None].astype(jnp.int32)
    idx_ref[0] = idx

    xnsq = ssq * (inv * inv)
    part = jnp.sum(xnsq - 2.0 * m + 1.0) * (1.0 / (_NROWS * _D))

    @pl.when(i == 0)
    def _():
        loss_ref[0, 0] = 0.0

    loss_ref[0, 0] += part


_tc_call = pl.pallas_call(
    _tc_body,
    grid=(_B,),
    in_specs=[
        pl.BlockSpec((1, _D, _BR), lambda i: (i, 0, 0)),
        pl.BlockSpec((_NE, _D), lambda i: (0, 0)),
    ],
    out_specs=[
        pl.BlockSpec((1, 1, _BR), lambda i: (i, 0, 0)),
        pl.BlockSpec(memory_space=pltpu.SMEM, block_shape=(1, 1),
                     index_map=lambda i: (0, 0)),
    ],
    out_shape=[
        jax.ShapeDtypeStruct((_B, 1, _BR), jnp.int32),
        jax.ShapeDtypeStruct((1, 1), jnp.float32),
    ],
    compiler_params=pltpu.CompilerParams(dimension_semantics=("arbitrary",)),
)


@functools.cache
def _sc_gather_call():
    @functools.partial(
        pl.kernel,
        mesh=plsc.VectorSubcoreMesh(core_axis_name="c", subcore_axis_name="s"),
        out_type=jax.ShapeDtypeStruct((_B, _D * _BR), jnp.float32),
        scratch_types=[
            pltpu.VMEM((_D, _NE), jnp.float32),
            pltpu.VMEM((_BR,), jnp.int32),
            pltpu.VMEM((_DCH * _BR,), jnp.float32),
        ],
        compiler_params=pltpu.CompilerParams(use_tc_tiling_on_sc=False,
                                             needs_layout_passes=False),
    )
    def _sc_gather(etf_hbm, idx_hbm, out_hbm, et_v, idx_v, q_v):
        b = lax.axis_index("s") * _NC + lax.axis_index("c")
        pltpu.sync_copy(etf_hbm, et_v)
        pltpu.sync_copy(idx_hbm.at[b], idx_v)

        for c in range(_D // _DCH):
            @plsc.parallel_loop(0, _BR // _L, 1, unroll=8)
            def _(r, c=c):
                base = r * _L
                cols = idx_v[pl.ds(base, _L)]
                for dd in range(_DCH):
                    rows = jnp.full((_L,), c * _DCH + dd, jnp.int32)
                    q_v[pl.ds(dd * _BR + base, _L)] = plsc.load_gather(
                        et_v, [rows, cols])

            pltpu.sync_copy(q_v, out_hbm.at[b, pl.ds(c * _DCH * _BR,
                                                     _DCH * _BR)])

    return _sc_gather


def kernel(inputs, embeddings):
    orig_shape = inputs.shape
    xt3 = jnp.swapaxes(inputs, 1, 2)
    idx3, loss11 = _tc_call(xt3, embeddings)
    idx2d = idx3.reshape(_B, _BR)
    et = embeddings.T
    qt = _sc_gather_call()(et, idx2d)
    quantized = jnp.swapaxes(qt.reshape(_B, _D, _BR), 1, 2).reshape(orig_shape)
    loss = loss11[0, 0]
    encoding_indices = idx2d.reshape(orig_shape[:-1])
    return (quantized, loss, encoding_indices)


# speedup vs baseline: 1.8302x; 1.1326x over previous
import functools

import jax
import jax.numpy as jnp
from jax import lax
from jax.experimental import pallas as pl
from jax.experimental.pallas import tpu as pltpu
from jax.experimental.pallas import tpu_sc as plsc

_NE = 1024
_D = 64
_B = 32
_BR = 1024
_NROWS = _B * _BR
_NC, _NS = 2, 16
_L = 16
_DCH = 16


def _tc_body(xt_ref, e_ref, idx_ref, loss_ref):
    i = pl.program_id(0)
    xt = xt_ref[0]
    ssq = jnp.sum(xt * xt, axis=0, keepdims=True)
    norm = jnp.sqrt(ssq)
    inv = 1.0 / jnp.maximum(norm, 1e-12)
    xnt = xt * inv

    s = lax.dot_general(e_ref[...], xnt, (((1,), (0,)), ((), ())),
                        preferred_element_type=jnp.float32)
    m = jnp.max(s, axis=0, keepdims=True)
    idx = jnp.argmax(s, axis=0)[None].astype(jnp.int32)
    idx_ref[0] = idx

    xnsq = ssq * (inv * inv)
    part = jnp.sum(xnsq - 2.0 * m + 1.0) * (1.0 / (_NROWS * _D))

    @pl.when(i == 0)
    def _():
        loss_ref[0, 0] = 0.0

    loss_ref[0, 0] += part


_tc_call = pl.pallas_call(
    _tc_body,
    grid=(_B,),
    in_specs=[
        pl.BlockSpec((1, _D, _BR), lambda i: (i, 0, 0)),
        pl.BlockSpec((_NE, _D), lambda i: (0, 0)),
    ],
    out_specs=[
        pl.BlockSpec((1, 1, _BR), lambda i: (i, 0, 0)),
        pl.BlockSpec(memory_space=pltpu.SMEM, block_shape=(1, 1),
                     index_map=lambda i: (0, 0)),
    ],
    out_shape=[
        jax.ShapeDtypeStruct((_B, 1, _BR), jnp.int32),
        jax.ShapeDtypeStruct((1, 1), jnp.float32),
    ],
    compiler_params=pltpu.CompilerParams(dimension_semantics=("arbitrary",)),
)


@functools.cache
def _sc_gather_call():
    @functools.partial(
        pl.kernel,
        mesh=plsc.VectorSubcoreMesh(core_axis_name="c", subcore_axis_name="s"),
        out_type=jax.ShapeDtypeStruct((_B, _D * _BR), jnp.float32),
        scratch_types=[
            pltpu.VMEM((_D * _NE,), jnp.float32),
            pltpu.VMEM((_BR,), jnp.int32),
            pltpu.VMEM((_DCH * _BR,), jnp.float32),
        ],
        compiler_params=pltpu.CompilerParams(use_tc_tiling_on_sc=False,
                                             needs_layout_passes=False),
    )
    def _sc_gather(etf_hbm, idx_hbm, out_hbm, et_v, idx_v, q_v):
        b = lax.axis_index("s") * _NC + lax.axis_index("c")
        pltpu.sync_copy(etf_hbm, et_v)
        pltpu.sync_copy(idx_hbm.at[b], idx_v)

        for c in range(_D // _DCH):
            @plsc.parallel_loop(0, _BR // _L, 1, unroll=8)
            def _(r, c=c):
                base = r * _L
                ct = base // 128
                cc = base % 128
                cols = idx_v[pl.ds(base, _L)]
                for dd in range(_DCH):
                    d = c * _DCH + dd
                    off = (dd // 8) * 8192 + (dd % 8) * 128 + cc
                    q_v[pl.ds(ct * 1024 + off, _L)] = plsc.load_gather(
                        et_v, [cols + d * _NE])

            pltpu.sync_copy(q_v, out_hbm.at[b, pl.ds(c * _DCH * _BR,
                                                     _DCH * _BR)])

    return _sc_gather


def kernel(inputs, embeddings):
    orig_shape = inputs.shape
    xt3 = jnp.swapaxes(inputs, 1, 2)
    idx3, loss11 = _tc_call(xt3, embeddings)
    idx2d = idx3.reshape(_B, _BR)
    etf = embeddings.T.reshape(-1)
    qt = _sc_gather_call()(etf, idx2d)
    quantized = jnp.transpose(qt.reshape(_B, 8, 8, 8, 128),
                              (0, 2, 4, 1, 3)).reshape(orig_shape)
    loss = loss11[0, 0]
    encoding_indices = idx2d.reshape(orig_shape[:-1])
    return (quantized, loss, encoding_indices)
